# baseline (speedup 1.0000x reference)
import functools

import jax
import jax.numpy as jnp
import numpy as np
from jax import lax
from jax.experimental import pallas as pl
from jax.experimental.pallas import tpu as pltpu
from jax.experimental.pallas import tpu_sc as plsc

_B, _K, _D = 128, 32768, 32
_EPS = np.float32(0.05)

_NC, _NS, _L = 2, 16, 16
_NW = _NC * _NS
_RPW = _B // _NW
_CH = 16384
_CPR = _K // _CH
_TOT = _RPW * _CPR
_NBUF = 6
_AHEAD = 3
_BLK = 8
_BLKE = _BLK * _L
_NIT = _CH // _BLKE
_IMAX = np.int32(2**31 - 1)

_GDN = lax.GatherDimensionNumbers(
    offset_dims=(), collapsed_slice_dims=(0,), start_index_map=(0,))


def _shuf(v, idx):
    return lax.gather(v, idx[:, None], dimension_numbers=_GDN,
                      slice_sizes=(1,),
                      mode=lax.GatherScatterMode.PROMISE_IN_BOUNDS)


@functools.partial(
    pl.kernel,
    out_type=jax.ShapeDtypeStruct((_NW, _L), jnp.int32),
    mesh=plsc.VectorSubcoreMesh(core_axis_name="c", subcore_axis_name="s"),
    compiler_params=pltpu.CompilerParams(needs_layout_passes=False),
    scratch_types=[
        pltpu.VMEM((_NBUF * _CH,), jnp.float32),
        pltpu.VMEM((_D,), jnp.int32),
        pltpu.VMEM((_B + _L,), jnp.int32),
        pltpu.VMEM((_B + _L,), jnp.float32),
        pltpu.VMEM((_L,), jnp.int32),
        pltpu.SemaphoreType.DMA,
        pltpu.SemaphoreType.DMA,
        pltpu.SemaphoreType.DMA,
        pltpu.SemaphoreType.DMA,
    ],
)
def _sc_sample(fhat_hbm, span_hbm, eidx_hbm, unif_hbm, out_hbm,
               ring, span_v, eidx_v, unif_v, res_v, s0, s1, s2, s3):
    sems = (s0, s1, s2, s3)
    cid = lax.axis_index("c")
    sid = lax.axis_index("s")
    wid = cid * _NS + sid
    row0 = wid * _RPW

    def fire(k):
        j, c = divmod(k, _CPR)
        cp = pltpu.make_async_copy(
            fhat_hbm.at[row0 + j, pl.ds(c * _CH, _CH)],
            ring.at[pl.ds((k % _NBUF) * _CH, _CH)],
            sems[k % _AHEAD])
        cp.start()
        return cp

    copies = {k: fire(k) for k in range(_AHEAD)}

    lane = lax.iota(jnp.int32, _L)
    neg = jnp.full((_L,), -jnp.inf, jnp.float32)
    zero = jnp.zeros((_L,), jnp.int32)

    ev = zero
    for j in range(_RPW):
        m, bi = neg, zero
        for c in range(_CPR):
            k = j * _CPR + c
            copies[k].wait()
            nk = k + _AHEAD
            if nk < _TOT:
                copies[nk] = fire(nk)
            off = (k % _NBUF) * _CH

            def bbody(i, carry, off=off, c=c):
                m, bi = carry
                vs = [ring[pl.ds(off + i * _BLKE + t * _L, _L)]
                      for t in range(_BLK)]
                while len(vs) > 1:
                    vs = [jnp.maximum(vs[t], vs[t + 1])
                          for t in range(0, len(vs), 2)]
                gt = vs[0] > m
                m = jnp.where(gt, vs[0], m)
                bi = jnp.where(gt, jnp.broadcast_to(i + c * _NIT, (_L,)), bi)
                return m, bi

            m, bi = lax.fori_loop(0, _NIT, bbody, (m, bi))

        mx = m
        for s in (8, 4, 2, 1):
            mx = jnp.maximum(mx, _shuf(mx, lane ^ s))
        cb = jnp.where(m == mx, bi, _IMAX)
        for s in (8, 4, 2, 1):
            cb = jnp.minimum(cb, _shuf(cb, lane ^ s))
        blk = cb[0]
        slot = (j * _CPR) % _NBUF + (blk // _NIT)
        base = slot * _CH + (blk % _NIT) * _BLKE
        idxv = jnp.broadcast_to(blk * _BLKE, (_L,)) + lane
        cand = jnp.full((_L,), _IMAX)
        for t in range(_BLK):
            v = ring[pl.ds(base + t * _L, _L)]
            cand = jnp.minimum(cand, jnp.where(v == mx, idxv + t * _L, _IMAX))
        for s in (8, 4, 2, 1):
            cand = jnp.minimum(cand, _shuf(cand, lane ^ s))
        ev = jnp.where(lane == j, cand, ev)

    pltpu.sync_copy(span_hbm, span_v)
    pltpu.sync_copy(eidx_hbm, eidx_v.at[pl.ds(0, _B)])
    pltpu.sync_copy(unif_hbm, unif_v.at[pl.ds(0, _B)])
    e16 = eidx_v[pl.ds(row0, _L)] & (_D - 1)
    u16 = unif_v[pl.ds(row0, _L)]
    ex16 = plsc.load_gather(span_v, [e16])
    res_v[...] = jnp.where(u16 < _EPS, ex16, ev)
    pltpu.sync_copy(res_v, out_hbm.at[wid])


def kernel(fhat, spanner, exploreindex, unif):
    out = _sc_sample(
        fhat,
        spanner.reshape(_D),
        exploreindex.reshape(_B),
        unif.reshape(_B),
    )
    return out[:, :_RPW].reshape(_B)


# speedup vs baseline: 1.0213x; 1.0213x over previous
import functools

import jax
import jax.numpy as jnp
import numpy as np
from jax import lax
from jax.experimental import pallas as pl
from jax.experimental.pallas import tpu as pltpu
from jax.experimental.pallas import tpu_sc as plsc

_B, _K, _D = 128, 32768, 32
_EPS = np.float32(0.05)

_NC, _NS, _L = 2, 16, 16
_NW = _NC * _NS
_RPW = _B // _NW
_CH = 16384
_CPR = _K // _CH
_TOT = _RPW * _CPR
_NBUF = 6
_AHEAD = 3
_BLK = 8
_BLKE = _BLK * _L
_NIT = _CH // _BLKE
_IMAX = np.int32(2**31 - 1)

_GDN = lax.GatherDimensionNumbers(
    offset_dims=(), collapsed_slice_dims=(0,), start_index_map=(0,))


def _shuf(v, idx):
    return lax.gather(v, idx[:, None], dimension_numbers=_GDN,
                      slice_sizes=(1,),
                      mode=lax.GatherScatterMode.PROMISE_IN_BOUNDS)


@functools.partial(
    pl.kernel,
    out_type=jax.ShapeDtypeStruct((_NW, _L), jnp.int32),
    mesh=plsc.VectorSubcoreMesh(core_axis_name="c", subcore_axis_name="s"),
    compiler_params=pltpu.CompilerParams(needs_layout_passes=False),
    scratch_types=[
        pltpu.VMEM((_NBUF * _CH,), jnp.float32),
        pltpu.VMEM((_D,), jnp.int32),
        pltpu.VMEM((_B + _L,), jnp.int32),
        pltpu.VMEM((_B + _L,), jnp.float32),
        pltpu.VMEM((_L,), jnp.int32),
        pltpu.SemaphoreType.DMA,
        pltpu.SemaphoreType.DMA,
        pltpu.SemaphoreType.DMA,
        pltpu.SemaphoreType.DMA,
    ],
)
def _sc_sample(fhat_hbm, span_hbm, eidx_hbm, unif_hbm, out_hbm,
               ring, span_v, eidx_v, unif_v, res_v, s0, s1, s2, s3):
    sems = (s0, s1, s2, s3)
    cid = lax.axis_index("c")
    sid = lax.axis_index("s")
    wid = cid * _NS + sid
    row0 = wid * _RPW

    def fire(k):
        j, c = divmod(k, _CPR)
        cp = pltpu.make_async_copy(
            fhat_hbm.at[row0 + j, pl.ds(c * _CH, _CH)],
            ring.at[pl.ds((k % _NBUF) * _CH, _CH)],
            sems[k % _AHEAD])
        cp.start()
        return cp

    copies = {k: fire(k) for k in range(_AHEAD)}

    pltpu.sync_copy(span_hbm, span_v)
    pltpu.sync_copy(eidx_hbm, eidx_v.at[pl.ds(0, _B)])
    pltpu.sync_copy(unif_hbm, unif_v.at[pl.ds(0, _B)])

    lane = lax.iota(jnp.int32, _L)
    neg = jnp.full((_L,), -jnp.inf, jnp.float32)
    zero = jnp.zeros((_L,), jnp.int32)

    ev = zero
    for j in range(_RPW):
        m, bi = neg, zero
        for c in range(_CPR):
            k = j * _CPR + c
            copies[k].wait()
            nk = k + _AHEAD
            if nk < _TOT:
                copies[nk] = fire(nk)
            off = (k % _NBUF) * _CH

            def bbody(i, carry, off=off, c=c):
                m, bi = carry
                vs = [ring[pl.ds(off + i * _BLKE + t * _L, _L)]
                      for t in range(_BLK)]
                while len(vs) > 1:
                    vs = [jnp.maximum(vs[t], vs[t + 1])
                          for t in range(0, len(vs), 2)]
                gt = vs[0] > m
                m = jnp.where(gt, vs[0], m)
                bi = jnp.where(gt, jnp.broadcast_to(i + c * _NIT, (_L,)), bi)
                return m, bi

            m, bi = lax.fori_loop(0, _NIT, bbody, (m, bi))

        mx = m
        for s in (8, 4, 2, 1):
            mx = jnp.maximum(mx, _shuf(mx, lane ^ s))
        cb = jnp.where(m == mx, bi, _IMAX)
        for s in (8, 4, 2, 1):
            cb = jnp.minimum(cb, _shuf(cb, lane ^ s))
        blk = cb[0]
        slot = (j * _CPR) % _NBUF + (blk // _NIT)
        base = slot * _CH + (blk % _NIT) * _BLKE
        idxv = jnp.broadcast_to(blk * _BLKE, (_L,)) + lane
        cand = jnp.full((_L,), _IMAX)
        for t in range(_BLK):
            v = ring[pl.ds(base + t * _L, _L)]
            cand = jnp.minimum(cand, jnp.where(v == mx, idxv + t * _L, _IMAX))
        for s in (8, 4, 2, 1):
            cand = jnp.minimum(cand, _shuf(cand, lane ^ s))
        ev = jnp.where(lane == j, cand, ev)

    e16 = eidx_v[pl.ds(row0, _L)] & (_D - 1)
    u16 = unif_v[pl.ds(row0, _L)]
    ex16 = plsc.load_gather(span_v, [e16])
    res_v[...] = jnp.where(u16 < _EPS, ex16, ev)
    pltpu.sync_copy(res_v, out_hbm.at[wid])


def kernel(fhat, spanner, exploreindex, unif):
    out = _sc_sample(
        fhat,
        spanner.reshape(_D),
        exploreindex.reshape(_B),
        unif.reshape(_B),
    )
    return out[:, :_RPW].reshape(_B)
